# Initial kernel scaffold; baseline (speedup 1.0000x reference)
import jax
import jax.numpy as jnp
from jax.experimental import pallas as pl


def kernel(nrow, x):
    raise NotImplementedError("write your pallas kernel here")


# speedup vs baseline: 1.8985x; 1.8985x over previous
import functools

import jax
import jax.numpy as jnp
from jax import lax
from jax.experimental import pallas as pl
from jax.experimental.pallas import tpu as pltpu
from jax.experimental.pallas import tpu_sc as plsc

N = 33554432
NROW = 65536
NC = 2
NS = 16
NW = NC * NS
L = 16
SHARD = N // NW
CHUNK = 16384
NCHUNK = SHARD // CHUNK
CBIN = NROW // NW
NVEC = CBIN // L

def _wid():
    return lax.axis_index("s") * NC + lax.axis_index("c")


def _hist_body(x_hbm, parts_hbm, sums_hbm, hist_v, buf0, buf1, sum_v, sem0, sem1):
    wid = _wid()
    base = wid * SHARD

    zeros16 = jnp.zeros((L,), jnp.int32)
    ones16 = jnp.ones((L,), jnp.int32)
    clamp16 = jnp.full((L,), NROW - 1, jnp.int32)

    def zbody(i, _):
        hist_v[pl.ds(i * L, L)] = zeros16
        return 0

    lax.fori_loop(0, NROW // L, zbody, 0)

    bufs = (buf0, buf1)
    sems = (sem0, sem1)
    handles = [None, None]
    handles[0] = pltpu.async_copy(x_hbm.at[pl.ds(base, CHUNK)], buf0, sem0)

    def process(buf):
        def ibody(k, _):
            idx = jnp.minimum(buf[pl.ds(k * L, L)], clamp16)
            plsc.addupdate_scatter(hist_v, [idx], ones16)
            return 0

        lax.fori_loop(0, CHUNK // L, ibody, 0)

    for ch in range(NCHUNK):
        b = ch & 1
        handles[b].wait()
        if ch + 1 < NCHUNK:
            handles[1 - b] = pltpu.async_copy(
                x_hbm.at[pl.ds(base + (ch + 1) * CHUNK, CHUNK)],
                bufs[1 - b],
                sems[1 - b],
            )
        process(bufs[b])

    lanes = lax.iota(jnp.int32, L)
    s0 = zeros16
    s1 = zeros16
    for cblk in range(NW):
        def sbody(j, acc):
            return acc + hist_v[pl.ds(cblk * CBIN + j * L, L)]

        tot = jnp.sum(lax.fori_loop(0, NVEC, sbody, zeros16))
        onehot = jnp.where(lanes == (cblk % L), tot, 0)
        if cblk < L:
            s0 = s0 + onehot
        else:
            s1 = s1 + onehot
    sum_v[pl.ds(0, L)] = s0
    sum_v[pl.ds(L, L)] = s1

    pltpu.sync_copy(hist_v, parts_hbm.at[pl.ds(wid * NROW, NROW)])
    pltpu.sync_copy(sum_v, sums_hbm.at[pl.ds(wid * NW, NW)])


def _scan_body(parts_hbm, sums_hbm, out_hbm, slab_v, sums_v, out_v, sem):
    wid = _wid()
    base = wid * CBIN

    zeros16 = jnp.zeros((L,), jnp.int32)

    handles = []
    for t in range(NW):
        handles.append(
            pltpu.async_copy(
                parts_hbm.at[pl.ds(t * NROW + base, CBIN)],
                slab_v.at[pl.ds(t * CBIN, CBIN)],
                sem,
            )
        )
    pltpu.sync_copy(sums_hbm, sums_v)

    def tbody(r, acc):
        t0, t1 = acc
        return (
            t0 + sums_v[pl.ds(r * NW, L)],
            t1 + sums_v[pl.ds(r * NW + L, L)],
        )

    t0, t1 = lax.fori_loop(0, NW, tbody, (zeros16, zeros16))
    lanes = lax.iota(jnp.int32, L)
    off = jnp.sum(jnp.where(lanes < wid, t0, 0)) + jnp.sum(
        jnp.where(lanes < wid - L, t1, 0)
    )

    for h in handles:
        h.wait()

    def jbody(j, carry):
        def tb(t, acc):
            return acc + slab_v[pl.ds(t * CBIN + j * L, L)]

        v = lax.fori_loop(0, NW, tb, zeros16)
        out_v[pl.ds(j * L, L)] = plsc.cumsum(v) + carry
        return carry + jnp.sum(v)

    lax.fori_loop(0, NVEC, jbody, off)
    pltpu.sync_copy(out_v, out_hbm.at[pl.ds(base, CBIN)])


@functools.cache
def _build():
    mesh = plsc.VectorSubcoreMesh(
        core_axis_name="c", subcore_axis_name="s", num_cores=NC, num_subcores=NS
    )
    params = pltpu.CompilerParams(needs_layout_passes=False)
    hist = pl.kernel(
        _hist_body,
        compiler_params=params,
        out_type=[
            jax.ShapeDtypeStruct((NW * NROW,), jnp.int32),
            jax.ShapeDtypeStruct((NW * NW,), jnp.int32),
        ],
        mesh=mesh,
        scratch_types=[
            pltpu.VMEM((NROW,), jnp.int32),
            pltpu.VMEM((CHUNK,), jnp.int32),
            pltpu.VMEM((CHUNK,), jnp.int32),
            pltpu.VMEM((NW,), jnp.int32),
            pltpu.SemaphoreType.DMA,
            pltpu.SemaphoreType.DMA,
        ],
    )
    scan = pl.kernel(
        _scan_body,
        compiler_params=params,
        out_type=jax.ShapeDtypeStruct((NROW,), jnp.int32),
        mesh=mesh,
        scratch_types=[
            pltpu.VMEM((NW * CBIN,), jnp.int32),
            pltpu.VMEM((NW * NW,), jnp.int32),
            pltpu.VMEM((CBIN,), jnp.int32),
            pltpu.SemaphoreType.DMA,
        ],
    )
    return hist, scan


def kernel(nrow, x):
    hist, scan = _build()
    parts, sums = hist(x)
    return scan(parts, sums)


# speedup vs baseline: 1.9692x; 1.0373x over previous
import functools

import jax
import jax.numpy as jnp
from jax import lax
from jax.experimental import pallas as pl
from jax.experimental.pallas import tpu as pltpu
from jax.experimental.pallas import tpu_sc as plsc

N = 33554432
NROW = 65536
NC = 2
NS = 16
NW = NC * NS
L = 16
SHARD = N // NW
CHUNK = 16384
NCHUNK = SHARD // CHUNK
CBIN = NROW // NW
NVEC = CBIN // L

def _wid():
    return lax.axis_index("s") * NC + lax.axis_index("c")


def _hist_body(x_hbm, parts_hbm, sums_hbm, hist_v, buf0, buf1, sum_v, sem0, sem1):
    wid = _wid()
    base = wid * SHARD

    zeros16 = jnp.zeros((L,), jnp.int32)
    ones16 = jnp.ones((L,), jnp.int32)
    clamp16 = jnp.full((L,), NROW - 1, jnp.int32)

    def zbody(i, _):
        hist_v[pl.ds(i * L, L)] = zeros16
        return 0

    lax.fori_loop(0, NROW // L, zbody, 0)

    bufs = (buf0, buf1)
    sems = (sem0, sem1)
    handles = [None, None]
    handles[0] = pltpu.async_copy(x_hbm.at[pl.ds(base, CHUNK)], buf0, sem0)

    UNROLL = 8

    def process(buf):
        def ibody(k, _):
            for u in range(UNROLL):
                idx = jnp.minimum(buf[pl.ds(k * (L * UNROLL) + u * L, L)], clamp16)
                plsc.addupdate_scatter(hist_v, [idx], ones16)
            return 0

        lax.fori_loop(0, CHUNK // (L * UNROLL), ibody, 0)

    for ch in range(NCHUNK):
        b = ch & 1
        handles[b].wait()
        if ch + 1 < NCHUNK:
            handles[1 - b] = pltpu.async_copy(
                x_hbm.at[pl.ds(base + (ch + 1) * CHUNK, CHUNK)],
                bufs[1 - b],
                sems[1 - b],
            )
        process(bufs[b])

    lanes = lax.iota(jnp.int32, L)
    s0 = zeros16
    s1 = zeros16
    for cblk in range(NW):
        def sbody(j, acc):
            return acc + hist_v[pl.ds(cblk * CBIN + j * L, L)]

        tot = jnp.sum(lax.fori_loop(0, NVEC, sbody, zeros16))
        onehot = jnp.where(lanes == (cblk % L), tot, 0)
        if cblk < L:
            s0 = s0 + onehot
        else:
            s1 = s1 + onehot
    sum_v[pl.ds(0, L)] = s0
    sum_v[pl.ds(L, L)] = s1

    pltpu.sync_copy(hist_v, parts_hbm.at[pl.ds(wid * NROW, NROW)])
    pltpu.sync_copy(sum_v, sums_hbm.at[pl.ds(wid * NW, NW)])


def _scan_body(parts_hbm, sums_hbm, out_hbm, slab_v, sums_v, out_v, sem):
    wid = _wid()
    base = wid * CBIN

    zeros16 = jnp.zeros((L,), jnp.int32)

    handles = []
    for t in range(NW):
        handles.append(
            pltpu.async_copy(
                parts_hbm.at[pl.ds(t * NROW + base, CBIN)],
                slab_v.at[pl.ds(t * CBIN, CBIN)],
                sem,
            )
        )
    pltpu.sync_copy(sums_hbm, sums_v)

    def tbody(r, acc):
        t0, t1 = acc
        return (
            t0 + sums_v[pl.ds(r * NW, L)],
            t1 + sums_v[pl.ds(r * NW + L, L)],
        )

    t0, t1 = lax.fori_loop(0, NW, tbody, (zeros16, zeros16))
    lanes = lax.iota(jnp.int32, L)
    off = jnp.sum(jnp.where(lanes < wid, t0, 0)) + jnp.sum(
        jnp.where(lanes < wid - L, t1, 0)
    )

    for h in handles:
        h.wait()

    def jbody(j, carry):
        def tb(t, acc):
            return acc + slab_v[pl.ds(t * CBIN + j * L, L)]

        v = lax.fori_loop(0, NW, tb, zeros16)
        out_v[pl.ds(j * L, L)] = plsc.cumsum(v) + carry
        return carry + jnp.sum(v)

    lax.fori_loop(0, NVEC, jbody, off)
    pltpu.sync_copy(out_v, out_hbm.at[pl.ds(base, CBIN)])


@functools.cache
def _build():
    mesh = plsc.VectorSubcoreMesh(
        core_axis_name="c", subcore_axis_name="s", num_cores=NC, num_subcores=NS
    )
    params = pltpu.CompilerParams(needs_layout_passes=False)
    hist = pl.kernel(
        _hist_body,
        compiler_params=params,
        out_type=[
            jax.ShapeDtypeStruct((NW * NROW,), jnp.int32),
            jax.ShapeDtypeStruct((NW * NW,), jnp.int32),
        ],
        mesh=mesh,
        scratch_types=[
            pltpu.VMEM((NROW,), jnp.int32),
            pltpu.VMEM((CHUNK,), jnp.int32),
            pltpu.VMEM((CHUNK,), jnp.int32),
            pltpu.VMEM((NW,), jnp.int32),
            pltpu.SemaphoreType.DMA,
            pltpu.SemaphoreType.DMA,
        ],
    )
    scan = pl.kernel(
        _scan_body,
        compiler_params=params,
        out_type=jax.ShapeDtypeStruct((NROW,), jnp.int32),
        mesh=mesh,
        scratch_types=[
            pltpu.VMEM((NW * CBIN,), jnp.int32),
            pltpu.VMEM((NW * NW,), jnp.int32),
            pltpu.VMEM((CBIN,), jnp.int32),
            pltpu.SemaphoreType.DMA,
        ],
    )
    return hist, scan


def kernel(nrow, x):
    hist, scan = _build()
    parts, sums = hist(x)
    return scan(parts, sums)


# speedup vs baseline: 3.4087x; 1.7310x over previous
import functools

import jax
import jax.numpy as jnp
from jax import lax
from jax.experimental import pallas as pl
from jax.experimental.pallas import tpu as pltpu
from jax.experimental.pallas import tpu_sc as plsc

N = 33554432
NROW = 65536
NC = 2
NS = 16
NW = NC * NS
L = 16
SHARD = N // NW
CHUNK = 8192
NIT = SHARD // (2 * CHUNK)
TEC_ELEMS = NIT * CHUNK
SLICE = NROW // NS
CBIN = NROW // NW
NVEC = CBIN // L

def _wid():
    return lax.axis_index("s") * NC + lax.axis_index("c")


def _hist_body(
    x_hbm, ones_hbm, parts_hbm, sums_hbm,
    hist_v, tbuf0, tbuf1, sbuf0, sbuf1, sbuf2, sbuf3, ones_v, sum_v, spmem,
    sem_t, sem_s, sem_sc,
):
    wid = _wid()
    sid = lax.axis_index("s")
    base = wid * SHARD
    sbase = base + TEC_ELEMS

    zeros16 = jnp.zeros((L,), jnp.int32)
    ones16 = jnp.ones((L,), jnp.int32)
    clamp16 = jnp.full((L,), NROW - 1, jnp.int32)

    tbufs = (tbuf0, tbuf1)
    sbufs = (sbuf0, sbuf1, sbuf2, sbuf3)
    tec_h = [None] * NIT
    stage_h = [None] * NIT
    scatter_h = [None] * NIT
    tec_h[0] = pltpu.async_copy(x_hbm.at[pl.ds(base, CHUNK)], tbuf0, sem_t)
    stage_h[0] = pltpu.async_copy(x_hbm.at[pl.ds(sbase, CHUNK)], sbuf0, sem_s)
    pltpu.sync_copy(ones_hbm, ones_v)

    def zbody(i, _):
        hist_v[pl.ds(i * L, L)] = zeros16
        return 0

    lax.fori_loop(0, NROW // L, zbody, 0)

    pltpu.sync_copy(hist_v.at[pl.ds(0, SLICE)], spmem.at[pl.ds(sid * SLICE, SLICE)])
    plsc.subcore_barrier()

    UNROLL = 8

    def process(buf):
        def ibody(k, _):
            for u in range(UNROLL):
                idx = jnp.minimum(buf[pl.ds(k * (L * UNROLL) + u * L, L)], clamp16)
                plsc.addupdate_scatter(hist_v, [idx], ones16)
            return 0

        lax.fori_loop(0, CHUNK // (L * UNROLL), ibody, 0)

    for i in range(NIT):
        stage_h[i].wait()
        scatter_h[i] = pltpu.async_copy(
            ones_v, spmem.at[sbufs[i & 3]], sem_sc, add=True
        )
        if i + 1 < NIT:
            if i - 3 >= 0:
                scatter_h[i - 3].wait()
            stage_h[i + 1] = pltpu.async_copy(
                x_hbm.at[pl.ds(sbase + (i + 1) * CHUNK, CHUNK)],
                sbufs[(i + 1) & 3],
                sem_s,
            )
        tec_h[i].wait()
        if i + 1 < NIT:
            tec_h[i + 1] = pltpu.async_copy(
                x_hbm.at[pl.ds(base + (i + 1) * CHUNK, CHUNK)],
                tbufs[(i + 1) & 1],
                sem_t,
            )
        process(tbufs[i & 1])

    for i in range(max(NIT - 4, 0), NIT):
        scatter_h[i].wait()
    plsc.subcore_barrier()

    pltpu.sync_copy(spmem.at[pl.ds(sid * SLICE, SLICE)], tbuf0.at[pl.ds(0, SLICE)])

    def mbody(j, _):
        o = sid * SLICE + j * L
        hist_v[pl.ds(o, L)] = hist_v[pl.ds(o, L)] + tbuf0[pl.ds(j * L, L)]
        return 0

    lax.fori_loop(0, SLICE // L, mbody, 0)

    lanes = lax.iota(jnp.int32, L)
    s0 = zeros16
    s1 = zeros16
    for cblk in range(NW):
        def sbody(j, acc):
            return acc + hist_v[pl.ds(cblk * CBIN + j * L, L)]

        tot = jnp.sum(lax.fori_loop(0, NVEC, sbody, zeros16))
        onehot = jnp.where(lanes == (cblk % L), tot, 0)
        if cblk < L:
            s0 = s0 + onehot
        else:
            s1 = s1 + onehot
    sum_v[pl.ds(0, L)] = s0
    sum_v[pl.ds(L, L)] = s1

    pltpu.sync_copy(hist_v, parts_hbm.at[pl.ds(wid * NROW, NROW)])
    pltpu.sync_copy(sum_v, sums_hbm.at[pl.ds(wid * NW, NW)])


def _scan_body(parts_hbm, sums_hbm, out_hbm, slab_v, sums_v, out_v, sem):
    wid = _wid()
    base = wid * CBIN

    zeros16 = jnp.zeros((L,), jnp.int32)

    handles = []
    for t in range(NW):
        handles.append(
            pltpu.async_copy(
                parts_hbm.at[pl.ds(t * NROW + base, CBIN)],
                slab_v.at[pl.ds(t * CBIN, CBIN)],
                sem,
            )
        )
    pltpu.sync_copy(sums_hbm, sums_v)

    def tbody(r, acc):
        t0, t1 = acc
        return (
            t0 + sums_v[pl.ds(r * NW, L)],
            t1 + sums_v[pl.ds(r * NW + L, L)],
        )

    t0, t1 = lax.fori_loop(0, NW, tbody, (zeros16, zeros16))
    lanes = lax.iota(jnp.int32, L)
    off = jnp.sum(jnp.where(lanes < wid, t0, 0)) + jnp.sum(
        jnp.where(lanes < wid - L, t1, 0)
    )

    for h in handles:
        h.wait()

    def jbody(j, carry):
        def tb(t, acc):
            return acc + slab_v[pl.ds(t * CBIN + j * L, L)]

        v = lax.fori_loop(0, NW, tb, zeros16)
        out_v[pl.ds(j * L, L)] = plsc.cumsum(v) + carry
        return carry + jnp.sum(v)

    lax.fori_loop(0, NVEC, jbody, off)
    pltpu.sync_copy(out_v, out_hbm.at[pl.ds(base, CBIN)])


@functools.cache
def _build():
    mesh = plsc.VectorSubcoreMesh(
        core_axis_name="c", subcore_axis_name="s", num_cores=NC, num_subcores=NS
    )
    params = pltpu.CompilerParams(needs_layout_passes=False)
    hist = pl.kernel(
        _hist_body,
        compiler_params=params,
        out_type=[
            jax.ShapeDtypeStruct((NW * NROW,), jnp.int32),
            jax.ShapeDtypeStruct((NW * NW,), jnp.int32),
        ],
        mesh=mesh,
        scratch_types=[
            pltpu.VMEM((NROW,), jnp.int32),
            pltpu.VMEM((CHUNK,), jnp.int32),
            pltpu.VMEM((CHUNK,), jnp.int32),
            pltpu.VMEM((CHUNK,), jnp.int32),
            pltpu.VMEM((CHUNK,), jnp.int32),
            pltpu.VMEM((CHUNK,), jnp.int32),
            pltpu.VMEM((CHUNK,), jnp.int32),
            pltpu.VMEM((CHUNK,), jnp.int32),
            pltpu.VMEM((NW,), jnp.int32),
            pltpu.VMEM_SHARED((NROW,), jnp.int32),
            pltpu.SemaphoreType.DMA,
            pltpu.SemaphoreType.DMA,
            pltpu.SemaphoreType.DMA,
        ],
    )
    scan = pl.kernel(
        _scan_body,
        compiler_params=params,
        out_type=jax.ShapeDtypeStruct((NROW,), jnp.int32),
        mesh=mesh,
        scratch_types=[
            pltpu.VMEM((NW * CBIN,), jnp.int32),
            pltpu.VMEM((NW * NW,), jnp.int32),
            pltpu.VMEM((CBIN,), jnp.int32),
            pltpu.SemaphoreType.DMA,
        ],
    )
    return hist, scan


def kernel(nrow, x):
    hist, scan = _build()
    ones = jnp.ones((CHUNK,), jnp.int32)
    parts, sums = hist(x, ones)
    return scan(parts, sums)
